# baseline (speedup 1.0000x reference)
import functools

import jax
import jax.numpy as jnp
from jax import lax
from jax.experimental import pallas as pl
from jax.experimental.pallas import tpu as pltpu
from jax.experimental.pallas import tpu_sc as plsc

N_V = 100000
N_E = 6400000
NC = 2
NS = 16
NW = NC * NS
L = 16

N_VP = 100096
E_PER_W = N_E // NW
CHUNK = 8000
N_CHUNKS = E_PER_W // CHUNK
INNER = CHUNK // L


@functools.partial(
    pl.kernel,
    mesh=plsc.VectorSubcoreMesh(core_axis_name="c", subcore_axis_name="s"),
    out_type=jax.ShapeDtypeStruct((NW * N_VP,), jnp.float32),
    scratch_types=[
        pltpu.VMEM((CHUNK,), jnp.int32),
        pltpu.VMEM((2 * CHUNK,), jnp.float32),
        pltpu.VMEM((N_VP,), jnp.float32),
    ],
    compiler_params=pltpu.CompilerParams(needs_layout_passes=False),
)
def _sc_scatter(eij, eattr, out, idx_ref, val_ref, acc_ref):
    c = lax.axis_index("c")
    s = lax.axis_index("s")
    wid = s * NC + c
    base = wid * E_PER_W

    zeros = jnp.zeros((L,), jnp.float32)

    def zbody(i, _):
        for k in range(16):
            acc_ref[pl.ds(i * 256 + k * L, L)] = zeros
        return 0

    lax.fori_loop(0, N_VP // 256, zbody, 0)

    iota2 = lax.iota(jnp.int32, L) * 2 + 1

    def chunk_body(j, _):
        off = base + j * CHUNK
        pltpu.sync_copy(eij.at[pl.ds(off, CHUNK)], idx_ref)
        pltpu.sync_copy(eattr.at[pl.ds(2 * off, 2 * CHUNK)], val_ref)

        def inner(i, _):
            rows = iota2 + i * (2 * L)
            idx = idx_ref[pl.ds(i * L, L)]
            vals = plsc.load_gather(val_ref, [rows])
            plsc.addupdate_scatter(acc_ref, [idx], vals)
            return 0

        lax.fori_loop(0, INNER, inner, 0)
        return 0

    lax.fori_loop(0, N_CHUNKS, chunk_body, 0)

    pltpu.sync_copy(acc_ref, out.at[pl.ds(wid * N_VP, N_VP)])


def _fin_body(va_ref, pp_ref, out_ref):
    b = va_ref[0, :]
    x = va_ref[1, :]
    cbar = pp_ref[0, :]
    for k in range(1, NW):
        cbar = cbar + pp_ref[k, :]
    out_ref[0, :] = b
    out_ref[1, :] = x
    out_ref[2, :] = b - cbar


def _finalize(va_p, pp):
    blk = N_VP // 2
    return pl.pallas_call(
        _fin_body,
        grid=(2,),
        in_specs=[
            pl.BlockSpec((2, blk), lambda j: (0, j)),
            pl.BlockSpec((NW, blk), lambda j: (0, j)),
        ],
        out_specs=pl.BlockSpec((3, blk), lambda j: (0, j)),
        out_shape=jax.ShapeDtypeStruct((3, N_VP), jnp.float32),
    )(va_p, pp)


@jax.jit
def kernel(vertex_attr, edgeij_pair, edge_attr, g, batch):
    pp = _sc_scatter(edgeij_pair.reshape(-1), edge_attr.reshape(-1))
    va_p = jnp.pad(vertex_attr.T, ((0, 0), (0, N_VP - N_V)))
    out_t = _finalize(va_p, pp.reshape(NW, N_VP))
    return out_t[:, :N_V].T


# speedup vs baseline: 34.4174x; 34.4174x over previous
import functools

import jax
import jax.numpy as jnp
from jax import lax
from jax.experimental import pallas as pl
from jax.experimental.pallas import tpu as pltpu
from jax.experimental.pallas import tpu_sc as plsc

N_V = 100000
N_E = 6400000
NC = 2
NS = 16
NW = NC * NS
L = 16

N_VP = 100352
CHUNK = 3200
N_CHUNKS = N_E // CHUNK
BASE_CHUNKS = N_CHUNKS // NW
EXTRA_TILES = N_CHUNKS - BASE_CHUNKS * NW
INNER = CHUNK // L


@functools.partial(
    pl.kernel,
    mesh=plsc.VectorSubcoreMesh(core_axis_name="c", subcore_axis_name="s"),
    out_type=jax.ShapeDtypeStruct((NW * N_VP,), jnp.float32),
    scratch_types=[
        pltpu.VMEM((2, CHUNK), jnp.int32),
        pltpu.VMEM((2, CHUNK), jnp.float32),
        pltpu.VMEM((N_VP,), jnp.float32),
    ],
    compiler_params=pltpu.CompilerParams(needs_layout_passes=False),
)
def _sc_scatter(eij, eattr_t, out, ij_ref, val_ref, acc_ref):
    c = lax.axis_index("c")
    s = lax.axis_index("s")
    wid = s * NC + c

    zeros = jnp.zeros((L,), jnp.float32)

    def zbody(i, _):
        for k in range(16):
            acc_ref[pl.ds(i * 256 + k * L, L)] = zeros
        return 0

    lax.fori_loop(0, N_VP // 256, zbody, 0)

    n_chunks = BASE_CHUNKS + jnp.where(wid < EXTRA_TILES, 1, 0)

    def chunk_body(j, _):
        off = (wid + j * NW) * CHUNK
        pltpu.sync_copy(eij.at[:, pl.ds(off, CHUNK)], ij_ref)
        pltpu.sync_copy(eattr_t.at[:, pl.ds(off, CHUNK)], val_ref)

        def inner(i, _):
            d = pl.ds(i * L, L)
            idx = ij_ref[0, d]
            vals = val_ref[1, d]
            plsc.addupdate_scatter(acc_ref, [idx], vals)
            return 0

        lax.fori_loop(0, INNER, inner, 0)
        return 0

    lax.fori_loop(0, n_chunks, chunk_body, 0)

    pltpu.sync_copy(acc_ref, out.at[pl.ds(wid * N_VP, N_VP)])


BLK = N_VP // 2


def _fin_body(va_ref, pp_ref, out_ref):
    k = pl.program_id(1)
    p = pp_ref[...]

    @pl.when(k == 0)
    def _():
        b = va_ref[0, :]
        out_ref[0, :] = b
        out_ref[1, :] = va_ref[1, :]
        out_ref[2, :] = b - p

    @pl.when(k != 0)
    def _():
        out_ref[2, :] = out_ref[2, :] - p


def _finalize(va_t, pp):
    return pl.pallas_call(
        _fin_body,
        grid=(N_VP // BLK, NW),
        in_specs=[
            pl.BlockSpec((2, BLK), lambda j, k: (0, j)),
            pl.BlockSpec((BLK,), lambda j, k: (k * (N_VP // BLK) + j,)),
        ],
        out_specs=pl.BlockSpec((3, BLK), lambda j, k: (0, j)),
        out_shape=jax.ShapeDtypeStruct((3, N_V), jnp.float32),
    )(va_t, pp)


@jax.jit
def kernel(vertex_attr, edgeij_pair, edge_attr, g, batch):
    pp = _sc_scatter(edgeij_pair, edge_attr.T)
    out_t = _finalize(vertex_attr.T, pp)
    return out_t.T


# speedup vs baseline: 66.6100x; 1.9354x over previous
import functools

import jax
import jax.numpy as jnp
from jax import lax
from jax.experimental import pallas as pl
from jax.experimental.pallas import tpu as pltpu
from jax.experimental.pallas import tpu_sc as plsc

N_V = 100000
N_E = 6400000
NC = 2
NS = 16
NW = NC * NS
L = 16

N_VP = 100352
CHUNK = 3200
N_CHUNKS = N_E // CHUNK
MAIN_J = 62
TAIL_START = MAIN_J * NW
INNER = CHUNK // L
U = 8


def _fire(eij, eattr_t, off, ij_r, val_r, sij, sval):
    pltpu.async_copy(eij.at[:, pl.ds(off, CHUNK)], ij_r, sij)
    pltpu.async_copy(eattr_t.at[:, pl.ds(off, CHUNK)], val_r, sval)


def _drain(eij, eattr_t, off, ij_r, val_r, sij, sval):
    pltpu.make_async_copy(eij.at[:, pl.ds(off, CHUNK)], ij_r, sij).wait()
    pltpu.make_async_copy(eattr_t.at[:, pl.ds(off, CHUNK)], val_r, sval).wait()


def _process(ij_r, val_r, acc_ref):
    def inner(i, _):
        for u in range(U):
            d = pl.ds((i * U + u) * L, L)
            plsc.addupdate_scatter(acc_ref, [ij_r[0, d]], val_r[1, d])
        return 0

    lax.fori_loop(0, INNER // U, inner, 0)


@functools.partial(
    pl.kernel,
    mesh=plsc.VectorSubcoreMesh(core_axis_name="c", subcore_axis_name="s"),
    out_type=jax.ShapeDtypeStruct((NW * N_VP,), jnp.float32),
    scratch_types=[
        pltpu.VMEM((2, CHUNK), jnp.int32),
        pltpu.VMEM((2, CHUNK), jnp.float32),
        pltpu.VMEM((2, CHUNK), jnp.int32),
        pltpu.VMEM((2, CHUNK), jnp.float32),
        pltpu.VMEM((N_VP,), jnp.float32),
        pltpu.SemaphoreType.DMA,
        pltpu.SemaphoreType.DMA,
        pltpu.SemaphoreType.DMA,
        pltpu.SemaphoreType.DMA,
    ],
    compiler_params=pltpu.CompilerParams(needs_layout_passes=False),
)
def _sc_scatter(eij, eattr_t, out, ij0, val0, ij1, val1, acc_ref,
                sij0, sval0, sij1, sval1):
    c = lax.axis_index("c")
    s = lax.axis_index("s")
    wid = s * NC + c

    def coff(j):
        return (wid + j * NW) * CHUNK

    _fire(eij, eattr_t, coff(0), ij0, val0, sij0, sval0)
    _fire(eij, eattr_t, coff(1), ij1, val1, sij1, sval1)

    zeros = jnp.zeros((L,), jnp.float32)

    def zbody(i, _):
        for k in range(16):
            acc_ref[pl.ds(i * 256 + k * L, L)] = zeros
        return 0

    lax.fori_loop(0, N_VP // 256, zbody, 0)

    last = MAIN_J - 1

    def pair(t, _):
        j0 = 2 * t
        _drain(eij, eattr_t, coff(j0), ij0, val0, sij0, sval0)
        _process(ij0, val0, acc_ref)
        _fire(eij, eattr_t, coff(jnp.minimum(j0 + 2, last)), ij0, val0,
              sij0, sval0)
        _drain(eij, eattr_t, coff(j0 + 1), ij1, val1, sij1, sval1)
        _process(ij1, val1, acc_ref)
        _fire(eij, eattr_t, coff(jnp.minimum(j0 + 3, last)), ij1, val1,
              sij1, sval1)
        return 0

    lax.fori_loop(0, MAIN_J // 2, pair, 0)

    _drain(eij, eattr_t, coff(last), ij0, val0, sij0, sval0)
    _drain(eij, eattr_t, coff(last), ij1, val1, sij1, sval1)

    @pl.when(wid < N_CHUNKS - TAIL_START)
    def _():
        off = (TAIL_START + wid) * CHUNK
        pltpu.sync_copy(eij.at[:, pl.ds(off, CHUNK)], ij0)
        pltpu.sync_copy(eattr_t.at[:, pl.ds(off, CHUNK)], val0)
        _process(ij0, val0, acc_ref)

    pltpu.sync_copy(acc_ref, out.at[pl.ds(wid * N_VP, N_VP)])


def _fin_body(va_ref, pp_ref, out_ref):
    k = pl.program_id(0)
    p = pp_ref[...]

    @pl.when(k == 0)
    def _():
        b = va_ref[0, :]
        out_ref[0, :] = b
        out_ref[1, :] = va_ref[1, :]
        out_ref[2, :] = b - p

    @pl.when(k != 0)
    def _():
        out_ref[2, :] = out_ref[2, :] - p


def _finalize(va_t, pp):
    return pl.pallas_call(
        _fin_body,
        grid=(NW,),
        in_specs=[
            pl.BlockSpec((2, N_VP), lambda k: (0, 0)),
            pl.BlockSpec((N_VP,), lambda k: (k,)),
        ],
        out_specs=pl.BlockSpec((3, N_VP), lambda k: (0, 0)),
        out_shape=jax.ShapeDtypeStruct((3, N_V), jnp.float32),
    )(va_t, pp)


@jax.jit
def kernel(vertex_attr, edgeij_pair, edge_attr, g, batch):
    pp = _sc_scatter(edgeij_pair, edge_attr.T)
    out_t = _finalize(vertex_attr.T, pp)
    return out_t.T


# speedup vs baseline: 73.1863x; 1.0987x over previous
import functools

import jax
import jax.numpy as jnp
from jax import lax
from jax.experimental import pallas as pl
from jax.experimental.pallas import tpu as pltpu
from jax.experimental.pallas import tpu_sc as plsc

N_V = 100000
N_E = 6400000
NC = 2
NS = 16
NW = NC * NS
L = 16

N_VP = 100352
CHUNK = 3200
N_CHUNKS = N_E // CHUNK
MAIN_J = 62
TAIL_START = MAIN_J * NW
INNER = CHUNK // L
U = 20


def _fire(eij, eattr_t, off, ij_r, val_r, sij, sval):
    pltpu.async_copy(eij.at[:, pl.ds(off, CHUNK)], ij_r, sij)
    pltpu.async_copy(eattr_t.at[:, pl.ds(off, CHUNK)], val_r, sval)


def _drain(eij, eattr_t, off, ij_r, val_r, sij, sval):
    pltpu.make_async_copy(eij.at[:, pl.ds(off, CHUNK)], ij_r, sij).wait()
    pltpu.make_async_copy(eattr_t.at[:, pl.ds(off, CHUNK)], val_r, sval).wait()


def _process(ij_r, val_r, acc_ref):
    def inner(i, _):
        for u in range(U):
            d = pl.ds((i * U + u) * L, L)
            plsc.addupdate_scatter(acc_ref, [ij_r[0, d]], val_r[1, d])
        return 0

    lax.fori_loop(0, INNER // U, inner, 0)


@functools.partial(
    pl.kernel,
    mesh=plsc.VectorSubcoreMesh(core_axis_name="c", subcore_axis_name="s"),
    out_type=jax.ShapeDtypeStruct((NW * N_VP,), jnp.float32),
    scratch_types=[
        pltpu.VMEM((2, CHUNK), jnp.int32),
        pltpu.VMEM((2, CHUNK), jnp.float32),
        pltpu.VMEM((2, CHUNK), jnp.int32),
        pltpu.VMEM((2, CHUNK), jnp.float32),
        pltpu.VMEM((N_VP,), jnp.float32),
        pltpu.SemaphoreType.DMA,
        pltpu.SemaphoreType.DMA,
        pltpu.SemaphoreType.DMA,
        pltpu.SemaphoreType.DMA,
    ],
    compiler_params=pltpu.CompilerParams(needs_layout_passes=False),
)
def _sc_scatter(eij, eattr_t, out, ij0, val0, ij1, val1, acc_ref,
                sij0, sval0, sij1, sval1):
    c = lax.axis_index("c")
    s = lax.axis_index("s")
    wid = s * NC + c

    def coff(j):
        return (wid + j * NW) * CHUNK

    _fire(eij, eattr_t, coff(0), ij0, val0, sij0, sval0)
    _fire(eij, eattr_t, coff(1), ij1, val1, sij1, sval1)

    zeros = jnp.zeros((L,), jnp.float32)

    def zbody(i, _):
        for k in range(16):
            acc_ref[pl.ds(i * 256 + k * L, L)] = zeros
        return 0

    lax.fori_loop(0, N_VP // 256, zbody, 0)

    last = MAIN_J - 1

    def pair(t, _):
        j0 = 2 * t
        _drain(eij, eattr_t, coff(j0), ij0, val0, sij0, sval0)
        _process(ij0, val0, acc_ref)
        _fire(eij, eattr_t, coff(jnp.minimum(j0 + 2, last)), ij0, val0,
              sij0, sval0)
        _drain(eij, eattr_t, coff(j0 + 1), ij1, val1, sij1, sval1)
        _process(ij1, val1, acc_ref)
        _fire(eij, eattr_t, coff(jnp.minimum(j0 + 3, last)), ij1, val1,
              sij1, sval1)
        return 0

    lax.fori_loop(0, MAIN_J // 2, pair, 0)

    _drain(eij, eattr_t, coff(last), ij0, val0, sij0, sval0)
    _drain(eij, eattr_t, coff(last), ij1, val1, sij1, sval1)

    @pl.when(wid < N_CHUNKS - TAIL_START)
    def _():
        off = (TAIL_START + wid) * CHUNK
        pltpu.sync_copy(eij.at[:, pl.ds(off, CHUNK)], ij0)
        pltpu.sync_copy(eattr_t.at[:, pl.ds(off, CHUNK)], val0)
        _process(ij0, val0, acc_ref)

    pltpu.sync_copy(acc_ref, out.at[pl.ds(wid * N_VP, N_VP)])


FIN_G = 4


def _fin_body(va_ref, pp_ref, out_ref):
    k = pl.program_id(0)
    p = pp_ref[pl.ds(0, N_VP)]
    for q in range(1, FIN_G):
        p = p + pp_ref[pl.ds(q * N_VP, N_VP)]

    @pl.when(k == 0)
    def _():
        b = va_ref[0, :]
        out_ref[0, :] = b
        out_ref[1, :] = va_ref[1, :]
        out_ref[2, :] = b - p

    @pl.when(k != 0)
    def _():
        out_ref[2, :] = out_ref[2, :] - p


def _finalize(va_t, pp):
    return pl.pallas_call(
        _fin_body,
        grid=(NW // FIN_G,),
        in_specs=[
            pl.BlockSpec((2, N_VP), lambda k: (0, 0)),
            pl.BlockSpec((FIN_G * N_VP,), lambda k: (k,)),
        ],
        out_specs=pl.BlockSpec((3, N_VP), lambda k: (0, 0)),
        out_shape=jax.ShapeDtypeStruct((3, N_V), jnp.float32),
    )(va_t, pp)


@jax.jit
def kernel(vertex_attr, edgeij_pair, edge_attr, g, batch):
    pp = _sc_scatter(edgeij_pair, edge_attr.T)
    out_t = _finalize(vertex_attr.T, pp)
    return out_t.T


# speedup vs baseline: 94.2240x; 1.2875x over previous
import functools

import jax
import jax.numpy as jnp
from jax import lax
from jax.experimental import pallas as pl
from jax.experimental.pallas import tpu as pltpu
from jax.experimental.pallas import tpu_sc as plsc

N_V = 100000
N_E = 6400000
NC = 2
NS = 16
NW = NC * NS
L = 16

N_VP = 100352
CHUNK = 3200
N_CHUNKS = N_E // CHUNK
MAIN_J = 62
TAIL_START = MAIN_J * NW
INNER = CHUNK // L
U = 20


def _fire(eij, eattr_t, off, ij_r, val_r, sij, sval):
    pltpu.async_copy(eij.at[:, pl.ds(off, CHUNK)], ij_r, sij)
    pltpu.async_copy(eattr_t.at[:, pl.ds(off, CHUNK)], val_r, sval)


def _drain(eij, eattr_t, off, ij_r, val_r, sij, sval):
    pltpu.make_async_copy(eij.at[:, pl.ds(off, CHUNK)], ij_r, sij).wait()
    pltpu.make_async_copy(eattr_t.at[:, pl.ds(off, CHUNK)], val_r, sval).wait()


def _process(ij_r, val_r, acc_ref):
    @plsc.parallel_loop(0, INNER, 1, unroll=U)
    def _(i):
        d = pl.ds(i * L, L)
        plsc.addupdate_scatter(acc_ref, [ij_r[0, d]], val_r[1, d])


@functools.partial(
    pl.kernel,
    mesh=plsc.VectorSubcoreMesh(core_axis_name="c", subcore_axis_name="s"),
    out_type=jax.ShapeDtypeStruct((NW * N_VP,), jnp.float32),
    scratch_types=[
        pltpu.VMEM((2, CHUNK), jnp.int32),
        pltpu.VMEM((2, CHUNK), jnp.float32),
        pltpu.VMEM((2, CHUNK), jnp.int32),
        pltpu.VMEM((2, CHUNK), jnp.float32),
        pltpu.VMEM((N_VP,), jnp.float32),
        pltpu.SemaphoreType.DMA,
        pltpu.SemaphoreType.DMA,
        pltpu.SemaphoreType.DMA,
        pltpu.SemaphoreType.DMA,
    ],
    compiler_params=pltpu.CompilerParams(needs_layout_passes=False),
)
def _sc_scatter(eij, eattr_t, out, ij0, val0, ij1, val1, acc_ref,
                sij0, sval0, sij1, sval1):
    c = lax.axis_index("c")
    s = lax.axis_index("s")
    wid = s * NC + c

    def coff(j):
        return (wid + j * NW) * CHUNK

    _fire(eij, eattr_t, coff(0), ij0, val0, sij0, sval0)
    _fire(eij, eattr_t, coff(1), ij1, val1, sij1, sval1)

    zeros = jnp.zeros((L,), jnp.float32)

    @plsc.parallel_loop(0, N_VP // L, 1, unroll=16)
    def _(i):
        acc_ref[pl.ds(i * L, L)] = zeros

    last = MAIN_J - 1

    def pair(t, _):
        j0 = 2 * t
        _drain(eij, eattr_t, coff(j0), ij0, val0, sij0, sval0)
        _process(ij0, val0, acc_ref)
        _fire(eij, eattr_t, coff(jnp.minimum(j0 + 2, last)), ij0, val0,
              sij0, sval0)
        _drain(eij, eattr_t, coff(j0 + 1), ij1, val1, sij1, sval1)
        _process(ij1, val1, acc_ref)
        _fire(eij, eattr_t, coff(jnp.minimum(j0 + 3, last)), ij1, val1,
              sij1, sval1)
        return 0

    lax.fori_loop(0, MAIN_J // 2, pair, 0)

    _drain(eij, eattr_t, coff(last), ij0, val0, sij0, sval0)
    _drain(eij, eattr_t, coff(last), ij1, val1, sij1, sval1)

    @pl.when(wid < N_CHUNKS - TAIL_START)
    def _():
        off = (TAIL_START + wid) * CHUNK
        pltpu.sync_copy(eij.at[:, pl.ds(off, CHUNK)], ij0)
        pltpu.sync_copy(eattr_t.at[:, pl.ds(off, CHUNK)], val0)
        _process(ij0, val0, acc_ref)

    pltpu.sync_copy(acc_ref, out.at[pl.ds(wid * N_VP, N_VP)])


FIN_G = 4


def _fin_body(va_ref, pp_ref, out_ref):
    k = pl.program_id(0)
    p = pp_ref[pl.ds(0, N_VP)]
    for q in range(1, FIN_G):
        p = p + pp_ref[pl.ds(q * N_VP, N_VP)]

    @pl.when(k == 0)
    def _():
        b = va_ref[0, :]
        out_ref[0, :] = b
        out_ref[1, :] = va_ref[1, :]
        out_ref[2, :] = b - p

    @pl.when(k != 0)
    def _():
        out_ref[2, :] = out_ref[2, :] - p


def _finalize(va_t, pp):
    return pl.pallas_call(
        _fin_body,
        grid=(NW // FIN_G,),
        in_specs=[
            pl.BlockSpec((2, N_VP), lambda k: (0, 0)),
            pl.BlockSpec((FIN_G * N_VP,), lambda k: (k,)),
        ],
        out_specs=pl.BlockSpec((3, N_VP), lambda k: (0, 0)),
        out_shape=jax.ShapeDtypeStruct((3, N_V), jnp.float32),
    )(va_t, pp)


@jax.jit
def kernel(vertex_attr, edgeij_pair, edge_attr, g, batch):
    pp = _sc_scatter(edgeij_pair, edge_attr.T)
    out_t = _finalize(vertex_attr.T, pp)
    return out_t.T


# speedup vs baseline: 96.0070x; 1.0189x over previous
import functools

import jax
import jax.numpy as jnp
from jax import lax
from jax.experimental import pallas as pl
from jax.experimental.pallas import tpu as pltpu
from jax.experimental.pallas import tpu_sc as plsc

N_V = 100000
N_E = 6400000
NC = 2
NS = 16
NW = NC * NS
L = 16

N_VP = 100352
CHUNK = 3200
N_CHUNKS = N_E // CHUNK
MAIN_J = 62
TAIL_START = MAIN_J * NW
INNER = CHUNK // L
U = 20


def _fire(eij, eattr_t, off, ij_r, val_r, sij, sval):
    pltpu.async_copy(eij.at[:, pl.ds(off, CHUNK)], ij_r, sij)
    pltpu.async_copy(eattr_t.at[:, pl.ds(off, CHUNK)], val_r, sval)


def _drain(eij, eattr_t, off, ij_r, val_r, sij, sval):
    pltpu.make_async_copy(eij.at[:, pl.ds(off, CHUNK)], ij_r, sij).wait()
    pltpu.make_async_copy(eattr_t.at[:, pl.ds(off, CHUNK)], val_r, sval).wait()


def _process(ij_r, val_r, acc_ref):
    @plsc.parallel_loop(0, INNER, 1, unroll=U)
    def _(i):
        d = pl.ds(i * L, L)
        plsc.addupdate_scatter(acc_ref, [ij_r[0, d]], val_r[1, d])


@functools.partial(
    pl.kernel,
    mesh=plsc.VectorSubcoreMesh(core_axis_name="c", subcore_axis_name="s"),
    out_type=jax.ShapeDtypeStruct((NW * N_VP,), jnp.float32),
    scratch_types=[
        pltpu.VMEM((2, CHUNK), jnp.int32),
        pltpu.VMEM((2, CHUNK), jnp.float32),
        pltpu.VMEM((2, CHUNK), jnp.int32),
        pltpu.VMEM((2, CHUNK), jnp.float32),
        pltpu.VMEM((N_VP,), jnp.float32),
        pltpu.SemaphoreType.DMA,
        pltpu.SemaphoreType.DMA,
        pltpu.SemaphoreType.DMA,
        pltpu.SemaphoreType.DMA,
    ],
    compiler_params=pltpu.CompilerParams(needs_layout_passes=False),
)
def _sc_scatter(eij, eattr_t, out, ij0, val0, ij1, val1, acc_ref,
                sij0, sval0, sij1, sval1):
    c = lax.axis_index("c")
    s = lax.axis_index("s")
    wid = s * NC + c

    def coff(j):
        return (wid + j * NW) * CHUNK

    _fire(eij, eattr_t, coff(0), ij0, val0, sij0, sval0)
    _fire(eij, eattr_t, coff(1), ij1, val1, sij1, sval1)

    zeros = jnp.zeros((L,), jnp.float32)

    @plsc.parallel_loop(0, N_VP // L, 1, unroll=16)
    def _(i):
        acc_ref[pl.ds(i * L, L)] = zeros

    last = MAIN_J - 1

    def pair(t, _):
        j0 = 2 * t
        _drain(eij, eattr_t, coff(j0), ij0, val0, sij0, sval0)
        _process(ij0, val0, acc_ref)
        _fire(eij, eattr_t, coff(jnp.minimum(j0 + 2, last)), ij0, val0,
              sij0, sval0)
        _drain(eij, eattr_t, coff(j0 + 1), ij1, val1, sij1, sval1)
        _process(ij1, val1, acc_ref)
        _fire(eij, eattr_t, coff(jnp.minimum(j0 + 3, last)), ij1, val1,
              sij1, sval1)
        return 0

    lax.fori_loop(0, MAIN_J // 2, pair, 0)

    _drain(eij, eattr_t, coff(last), ij0, val0, sij0, sval0)
    _drain(eij, eattr_t, coff(last), ij1, val1, sij1, sval1)

    @pl.when(wid < N_CHUNKS - TAIL_START)
    def _():
        off = (TAIL_START + wid) * CHUNK
        pltpu.sync_copy(eij.at[:, pl.ds(off, CHUNK)], ij0)
        pltpu.sync_copy(eattr_t.at[:, pl.ds(off, CHUNK)], val0)
        _process(ij0, val0, acc_ref)

    pltpu.sync_copy(acc_ref, out.at[pl.ds(wid * N_VP, N_VP)])


FIN_G = 8


def _fin_body(va_ref, pp_ref, out_ref):
    k = pl.program_id(0)
    p = pp_ref[pl.ds(0, N_VP)]
    for q in range(1, FIN_G):
        p = p + pp_ref[pl.ds(q * N_VP, N_VP)]

    @pl.when(k == 0)
    def _():
        b = va_ref[0, :]
        out_ref[0, :] = b
        out_ref[1, :] = va_ref[1, :]
        out_ref[2, :] = b - p

    @pl.when(k != 0)
    def _():
        out_ref[2, :] = out_ref[2, :] - p


def _finalize(va_t, pp):
    return pl.pallas_call(
        _fin_body,
        grid=(NW // FIN_G,),
        in_specs=[
            pl.BlockSpec((2, N_VP), lambda k: (0, 0)),
            pl.BlockSpec((FIN_G * N_VP,), lambda k: (k,)),
        ],
        out_specs=pl.BlockSpec((3, N_VP), lambda k: (0, 0)),
        out_shape=jax.ShapeDtypeStruct((3, N_V), jnp.float32),
    )(va_t, pp)


@jax.jit
def kernel(vertex_attr, edgeij_pair, edge_attr, g, batch):
    pp = _sc_scatter(edgeij_pair, edge_attr.T)
    out_t = _finalize(vertex_attr.T, pp)
    return out_t.T


# speedup vs baseline: 96.3650x; 1.0037x over previous
import functools

import jax
import jax.numpy as jnp
from jax import lax
from jax.experimental import pallas as pl
from jax.experimental.pallas import tpu as pltpu
from jax.experimental.pallas import tpu_sc as plsc

N_V = 100000
N_E = 6400000
NC = 2
NS = 16
NW = NC * NS
L = 16

N_VP = 100352
CHUNK = 3200
N_CHUNKS = N_E // CHUNK
MAIN_J = 62
TAIL_START = MAIN_J * NW
INNER = CHUNK // L
U = 20


def _fire(eij, eattr_t, off, ij_r, val_r, sij, sval):
    pltpu.async_copy(eij.at[:, pl.ds(off, CHUNK)], ij_r, sij)
    pltpu.async_copy(eattr_t.at[:, pl.ds(off, CHUNK)], val_r, sval)


def _drain(eij, eattr_t, off, ij_r, val_r, sij, sval):
    pltpu.make_async_copy(eij.at[:, pl.ds(off, CHUNK)], ij_r, sij).wait()
    pltpu.make_async_copy(eattr_t.at[:, pl.ds(off, CHUNK)], val_r, sval).wait()


def _process(ij_r, val_r, acc_ref):
    @plsc.parallel_loop(0, INNER, 1, unroll=U)
    def _(i):
        d = pl.ds(i * L, L)
        plsc.addupdate_scatter(acc_ref, [ij_r[0, d]], val_r[1, d])


@functools.partial(
    pl.kernel,
    mesh=plsc.VectorSubcoreMesh(core_axis_name="c", subcore_axis_name="s"),
    out_type=jax.ShapeDtypeStruct((NW * N_VP,), jnp.float32),
    scratch_types=[
        pltpu.VMEM((2, CHUNK), jnp.int32),
        pltpu.VMEM((2, CHUNK), jnp.float32),
        pltpu.VMEM((2, CHUNK), jnp.int32),
        pltpu.VMEM((2, CHUNK), jnp.float32),
        pltpu.VMEM((N_VP,), jnp.float32),
        pltpu.SemaphoreType.DMA,
        pltpu.SemaphoreType.DMA,
        pltpu.SemaphoreType.DMA,
        pltpu.SemaphoreType.DMA,
    ],
    compiler_params=pltpu.CompilerParams(needs_layout_passes=False),
)
def _sc_scatter(eij, eattr_t, out, ij0, val0, ij1, val1, acc_ref,
                sij0, sval0, sij1, sval1):
    c = lax.axis_index("c")
    s = lax.axis_index("s")
    wid = s * NC + c

    def coff(j):
        return (wid + j * NW) * CHUNK

    _fire(eij, eattr_t, coff(0), ij0, val0, sij0, sval0)
    _fire(eij, eattr_t, coff(1), ij1, val1, sij1, sval1)

    zeros = jnp.zeros((L,), jnp.float32)

    @plsc.parallel_loop(0, N_VP // L, 1, unroll=16)
    def _(i):
        acc_ref[pl.ds(i * L, L)] = zeros

    last = MAIN_J - 1

    def pair(t, _):
        j0 = 2 * t
        _drain(eij, eattr_t, coff(j0), ij0, val0, sij0, sval0)
        _process(ij0, val0, acc_ref)
        _fire(eij, eattr_t, coff(jnp.minimum(j0 + 2, last)), ij0, val0,
              sij0, sval0)
        _drain(eij, eattr_t, coff(j0 + 1), ij1, val1, sij1, sval1)
        _process(ij1, val1, acc_ref)
        _fire(eij, eattr_t, coff(jnp.minimum(j0 + 3, last)), ij1, val1,
              sij1, sval1)
        return 0

    lax.fori_loop(0, MAIN_J // 2, pair, 0)

    _drain(eij, eattr_t, coff(last), ij0, val0, sij0, sval0)
    _drain(eij, eattr_t, coff(last), ij1, val1, sij1, sval1)

    @pl.when(wid < N_CHUNKS - TAIL_START)
    def _():
        off = (TAIL_START + wid) * CHUNK
        pltpu.sync_copy(eij.at[:, pl.ds(off, CHUNK)], ij0)
        pltpu.sync_copy(eattr_t.at[:, pl.ds(off, CHUNK)], val0)
        _process(ij0, val0, acc_ref)

    pltpu.sync_copy(acc_ref, out.at[pl.ds(wid * N_VP, N_VP)])


FIN_G = 16


def _fin_body(va_ref, pp_ref, out_ref):
    k = pl.program_id(0)
    p = pp_ref[pl.ds(0, N_VP)]
    for q in range(1, FIN_G):
        p = p + pp_ref[pl.ds(q * N_VP, N_VP)]

    @pl.when(k == 0)
    def _():
        b = va_ref[0, :]
        out_ref[0, :] = b
        out_ref[1, :] = va_ref[1, :]
        out_ref[2, :] = b - p

    @pl.when(k != 0)
    def _():
        out_ref[2, :] = out_ref[2, :] - p


def _finalize(va_t, pp):
    return pl.pallas_call(
        _fin_body,
        grid=(NW // FIN_G,),
        in_specs=[
            pl.BlockSpec((2, N_VP), lambda k: (0, 0)),
            pl.BlockSpec((FIN_G * N_VP,), lambda k: (k,)),
        ],
        out_specs=pl.BlockSpec((3, N_VP), lambda k: (0, 0)),
        out_shape=jax.ShapeDtypeStruct((3, N_V), jnp.float32),
    )(va_t, pp)


@jax.jit
def kernel(vertex_attr, edgeij_pair, edge_attr, g, batch):
    pp = _sc_scatter(edgeij_pair, edge_attr.T)
    out_t = _finalize(vertex_attr.T, pp)
    return out_t.T
